# baseline (speedup 1.0000x reference)
import jax
import jax.numpy as jnp
from jax import lax
from jax.experimental import pallas as pl
from jax.experimental.pallas import tpu as pltpu

_TOTAL = 32768
_D = 128
_NSEG = 16
_BLK = 1024
_NBLK = _TOTAL // _BLK


def _body(cu_ref, flat_ref, w_ref, out_ref, acc_ref, mw_ref, cache_ref):
    p = pl.program_id(0)
    b = pl.program_id(1)

    starts = cu_ref[0:1, 0:_NSEG]
    ends = cu_ref[0:1, 1:_NSEG + 1]
    rows = lax.broadcasted_iota(jnp.int32, (_BLK, _NSEG), 0) + b * _BLK
    onehot = ((rows >= starts) & (rows < ends)).astype(jnp.float32)

    @pl.when((p == 0) & (b == 0))
    def _init():
        acc_ref[...] = jnp.zeros_like(acc_ref)

    @pl.when(p == 0)
    def _phase0():
        blk = flat_ref[...]
        cache_ref[pl.ds(b * _BLK, _BLK), :] = blk
        acc_ref[...] += lax.dot_general(
            onehot, blk, (((0,), (0,)), ((), ())),
            preferred_element_type=jnp.float32)

    @pl.when((p == 1) & (b == 0))
    def _means():
        mw_ref[...] = jnp.dot(acc_ref[...], w_ref[...],
                              preferred_element_type=jnp.float32)

    @pl.when(p == 1)
    def _phase1():
        inv = 1.0 / jnp.maximum((ends - starts).astype(jnp.float32), 1.0)
        blk = cache_ref[pl.ds(b * _BLK, _BLK), :]
        out_ref[...] = (
            jnp.dot(blk, w_ref[...], preferred_element_type=jnp.float32)
            - jnp.dot(onehot * inv, mw_ref[...],
                      preferred_element_type=jnp.float32))


def kernel(flat, cu_seqlens, W):
    cu2d = jnp.zeros((8, 128), jnp.int32).at[0, :_NSEG + 1].set(cu_seqlens)
    return pl.pallas_call(
        _body,
        grid=(2, _NBLK),
        in_specs=[
            pl.BlockSpec((8, 128), lambda p, b: (0, 0)),
            pl.BlockSpec((_BLK, _D), lambda p, b: (b * (1 - p), 0)),
            pl.BlockSpec((_D, _D), lambda p, b: (0, 0)),
        ],
        out_specs=pl.BlockSpec((_BLK, _D), lambda p, b: (b * p, 0)),
        out_shape=jax.ShapeDtypeStruct((_TOTAL, _D), jnp.float32),
        scratch_shapes=[
            pltpu.VMEM((_NSEG, _D), jnp.float32),
            pltpu.VMEM((_NSEG, _D), jnp.float32),
            pltpu.VMEM((_TOTAL, _D), jnp.float32),
        ],
        compiler_params=pltpu.CompilerParams(
            dimension_semantics=("arbitrary", "arbitrary"),
        ),
    )(cu2d, flat, W)


# speedup vs baseline: 2.2764x; 2.2764x over previous
import jax
import jax.numpy as jnp
from jax import lax
from jax.experimental import pallas as pl
from jax.experimental.pallas import tpu as pltpu

_TOTAL = 32768
_D = 128
_NSEG = 16
_BLK = 8192
_NBLK = _TOTAL // _BLK


def _body(bounds_ref, flat_ref, w_ref, out_ref, acc_ref, mw_ref):
    p = pl.program_id(0)
    b = pl.program_id(1)

    base = b * _BLK
    rows = bounds_ref[0:_NSEG, :] + base
    starts = bounds_ref[_NSEG:2 * _NSEG, :]
    ends = bounds_ref[2 * _NSEG:3 * _NSEG, :]
    onehot_t = ((rows >= starts) & (rows < ends)).astype(jnp.float32)

    @pl.when((p == 0) & (b == 0))
    def _init():
        acc_ref[...] = jnp.zeros_like(acc_ref)

    @pl.when(p == 0)
    def _phase0():
        acc_ref[...] += lax.dot_general(
            onehot_t, flat_ref[...], (((1,), (0,)), ((), ())),
            preferred_element_type=jnp.float32)

    @pl.when((p == 1) & (b == 0))
    def _means():
        counts = (bounds_ref[2 * _NSEG:3 * _NSEG, 0:_D]
                  - bounds_ref[_NSEG:2 * _NSEG, 0:_D]).astype(jnp.float32)
        mean = acc_ref[...] / jnp.maximum(counts, 1.0)
        mw_ref[...] = jnp.dot(mean, w_ref[...],
                              preferred_element_type=jnp.float32)

    @pl.when(p == 1)
    def _phase1():
        corr = lax.dot_general(
            onehot_t, mw_ref[...], (((0,), (0,)), ((), ())),
            preferred_element_type=jnp.float32)
        out_ref[...] = (
            jnp.dot(flat_ref[...], w_ref[...],
                    preferred_element_type=jnp.float32)
            - corr)


def kernel(flat, cu_seqlens, W):
    rows_base = jax.lax.broadcasted_iota(jnp.int32, (_NSEG, _BLK), 1)
    starts_b = jnp.broadcast_to(cu_seqlens[:_NSEG, None], (_NSEG, _BLK))
    ends_b = jnp.broadcast_to(cu_seqlens[1:_NSEG + 1, None], (_NSEG, _BLK))
    bounds = jnp.concatenate([rows_base, starts_b, ends_b], axis=0)
    return pl.pallas_call(
        _body,
        grid=(2, _NBLK),
        in_specs=[
            pl.BlockSpec((3 * _NSEG, _BLK), lambda p, b: (0, 0)),
            pl.BlockSpec((_BLK, _D), lambda p, b: (b, 0)),
            pl.BlockSpec((_D, _D), lambda p, b: (0, 0)),
        ],
        out_specs=pl.BlockSpec((_BLK, _D), lambda p, b: (b * p, 0)),
        out_shape=jax.ShapeDtypeStruct((_TOTAL, _D), jnp.float32),
        scratch_shapes=[
            pltpu.VMEM((_NSEG, _D), jnp.float32),
            pltpu.VMEM((_NSEG, _D), jnp.float32),
        ],
        compiler_params=pltpu.CompilerParams(
            dimension_semantics=("arbitrary", "arbitrary"),
        ),
    )(bounds, flat, W)


# speedup vs baseline: 2.3757x; 1.0436x over previous
import jax
import jax.numpy as jnp
from jax import lax
from jax.experimental import pallas as pl
from jax.experimental.pallas import tpu as pltpu

_TOTAL = 32768
_D = 128
_NSEG = 16
_BLK = 16384
_NBLK = _TOTAL // _BLK


def _body(bounds_ref, flat_ref, w_ref, out_ref, acc_ref, mw_ref):
    p = pl.program_id(0)
    b = pl.program_id(1)

    base = b * _BLK
    rows = bounds_ref[0:_NSEG, :] + base
    starts = bounds_ref[_NSEG:2 * _NSEG, :]
    ends = bounds_ref[2 * _NSEG:3 * _NSEG, :]
    onehot_t = ((rows >= starts) & (rows < ends)).astype(jnp.float32)

    @pl.when((p == 0) & (b == 0))
    def _init():
        acc_ref[...] = jnp.zeros_like(acc_ref)

    @pl.when(p == 0)
    def _phase0():
        acc_ref[...] += lax.dot_general(
            onehot_t, flat_ref[...], (((1,), (0,)), ((), ())),
            preferred_element_type=jnp.float32)

    @pl.when((p == 1) & (b == 0))
    def _means():
        counts = (bounds_ref[2 * _NSEG:3 * _NSEG, 0:_D]
                  - bounds_ref[_NSEG:2 * _NSEG, 0:_D]).astype(jnp.float32)
        mean = acc_ref[...] / jnp.maximum(counts, 1.0)
        mw_ref[...] = jnp.dot(mean, w_ref[...],
                              preferred_element_type=jnp.float32)

    @pl.when(p == 1)
    def _phase1():
        corr = lax.dot_general(
            onehot_t, mw_ref[...], (((0,), (0,)), ((), ())),
            preferred_element_type=jnp.float32)
        out_ref[...] = (
            jnp.dot(flat_ref[...], w_ref[...],
                    preferred_element_type=jnp.float32)
            - corr)


def kernel(flat, cu_seqlens, W):
    rows_base = jax.lax.broadcasted_iota(jnp.int32, (_NSEG, _BLK), 1)
    starts_b = jnp.broadcast_to(cu_seqlens[:_NSEG, None], (_NSEG, _BLK))
    ends_b = jnp.broadcast_to(cu_seqlens[1:_NSEG + 1, None], (_NSEG, _BLK))
    bounds = jnp.concatenate([rows_base, starts_b, ends_b], axis=0)
    return pl.pallas_call(
        _body,
        grid=(2, _NBLK),
        in_specs=[
            pl.BlockSpec((3 * _NSEG, _BLK), lambda p, b: (0, 0)),
            pl.BlockSpec((_BLK, _D), lambda p, b: (b, 0)),
            pl.BlockSpec((_D, _D), lambda p, b: (0, 0)),
        ],
        out_specs=pl.BlockSpec((_BLK, _D), lambda p, b: (b * p, 0)),
        out_shape=jax.ShapeDtypeStruct((_TOTAL, _D), jnp.float32),
        scratch_shapes=[
            pltpu.VMEM((_NSEG, _D), jnp.float32),
            pltpu.VMEM((_NSEG, _D), jnp.float32),
        ],
        compiler_params=pltpu.CompilerParams(
            dimension_semantics=("arbitrary", "arbitrary"),
        ),
    )(bounds, flat, W)
